# baseline (speedup 1.0000x reference)
import functools

import jax
import jax.numpy as jnp
from jax.experimental import pallas as pl
from jax.experimental.pallas import tpu as pltpu

B, M, E, IN, Q, OUT, H = 128, 2048, 128, 512, 512, 512, 64
BB = 8
LOGM = 11


def _xor_partner(x, d):
  right = pltpu.roll(x, d, 1)
  left = pltpu.roll(x, M - d, 1)
  lane = jax.lax.broadcasted_iota(jnp.int32, x.shape, 1)
  bit_clear = (lane & d) == 0
  return jnp.where(bit_clear, left, right)


def _bitonic(keys, payload, lex_idx=None):
  lane = jax.lax.broadcasted_iota(jnp.int32, keys.shape, 1)
  for k in range(1, LOGM + 1):
    for j in range(k - 1, -1, -1):
      d = 1 << j
      ok = _xor_partner(keys, d)
      op = _xor_partner(payload, d)
      if lex_idx is not None:
        oi = _xor_partner(lex_idx, d)
        less = (keys < ok) | ((keys == ok) & (lex_idx < oi))
      else:
        less = keys < ok
      lower = (lane & d) == 0
      asc = (lane & (1 << k)) == 0
      take_self = less ^ lower ^ asc
      keys = jnp.where(take_self, keys, ok)
      payload = jnp.where(take_self, payload, op)
      if lex_idx is not None:
        lex_idx = jnp.where(take_self, lex_idx, oi)
  return keys, payload, lex_idx


def _prefix_prod(x):
  lane = jax.lax.broadcasted_iota(jnp.int32, x.shape, 1)
  for k in range(LOGM):
    s = 1 << k
    shifted = pltpu.roll(x, s, 1)
    x = x * jnp.where(lane >= s, shifted, 1.0)
  return x


def _block_kernel(rw_ref, ww_ref, us_ref, fg_ref, inp_ref, qry_ref, mem_ref,
                  iw_ref, ib_ref, w1w_ref, w1b_ref, w2w_ref, w2b_ref,
                  r1w_ref, r1b_ref, r2w_ref, r2b_ref, ow_ref, ob_ref,
                  wr_out, w_out, u_out, y_out, rmem_out):
  f32 = jnp.float32
  dot = functools.partial(jnp.dot, preferred_element_type=f32)

  inp = inp_ref[...]
  qry = qry_ref[...]
  mem = mem_ref[...]

  e_vec = dot(inp, iw_ref[...].T) + ib_ref[...][None, :]
  wi = dot(e_vec, w1w_ref[...].T) + w1b_ref[...][None, :]
  cvec = dot(wi, w2w_ref[...])
  cconst = dot(wi, w2b_ref[...][:, None])
  rq = dot(qry, r1w_ref[...].T) + r1b_ref[...][None, :]
  v = dot(rq, r2w_ref[...])
  rconst = dot(rq, r2b_ref[...][:, None])

  mul = jnp.sum(mem * cvec[:, None, :], axis=2) + cconst
  mul2m = jnp.sum(mem * v[:, None, :], axis=2)

  w_content = jax.nn.softmax(mul, axis=1)

  us = 1e-05 + (1.0 - 1e-05) * us_ref[...]
  ww = ww_ref[...]
  u = us + ww - us * ww
  u = u * (1.0 - fg_ref[...] * rw_ref[...])
  u_out[...] = u

  idx = jax.lax.broadcasted_iota(jnp.int32, u.shape, 1)
  su, phi, _ = _bitonic(u, idx.astype(f32), lex_idx=idx)
  alloc_sorted = (1.0 - su) * _prefix_prod(su)
  _, alloc, _ = _bitonic(phi, alloc_sorted)
  w = (w_content + alloc) * 0.5
  w_out[...] = w

  ev = jnp.sum(e_vec * v, axis=1, keepdims=True)
  mul2 = mul2m + w * ev + rconst
  wr = jax.nn.softmax(mul2, axis=1)
  wr_out[...] = wr

  rmem = mem + w[:, :, None] * e_vec[:, None, :]
  rmem_out[...] = rmem
  outv = jnp.sum(wr[:, :, None] * rmem, axis=1)
  y_out[...] = dot(outv, ow_ref[...].T) + ob_ref[...][None, :]


def _full(shape):
  nd = len(shape)
  return pl.BlockSpec(shape, lambda i: (0,) * nd)


def _blk(shape):
  nd = len(shape)
  return pl.BlockSpec(shape, lambda i: (i,) + (0,) * (nd - 1))


@jax.jit
def kernel(state_vector, free_gates, input, query, memory, I_w, I_b, W1_w,
           W1_b, W2_w, W2_b, R1_w, R1_b, R2_w, R2_b, O_w, O_b):
  rw, ww, us = state_vector[0], state_vector[1], state_vector[2]
  inp = input[:, 0, :]
  qry = query[:, 0, :]

  grid = (B // BB,)
  out_shapes = (
      jax.ShapeDtypeStruct((B, M), jnp.float32),
      jax.ShapeDtypeStruct((B, M), jnp.float32),
      jax.ShapeDtypeStruct((B, M), jnp.float32),
      jax.ShapeDtypeStruct((B, OUT), jnp.float32),
      jax.ShapeDtypeStruct((B, M, E), jnp.float32),
  )
  in_specs = [
      _blk((BB, M)), _blk((BB, M)), _blk((BB, M)),
      _blk((BB, 1)),
      _blk((BB, IN)), _blk((BB, Q)),
      _blk((BB, M, E)),
      _full((E, IN)), _full((E,)),
      _full((H, E)), _full((H,)),
      _full((H, E)), _full((H,)),
      _full((H, Q)), _full((H,)),
      _full((H, E)), _full((H,)),
      _full((OUT, E)), _full((OUT,)),
  ]
  out_specs = (
      _blk((BB, M)), _blk((BB, M)), _blk((BB, M)),
      _blk((BB, OUT)),
      _blk((BB, M, E)),
  )
  wr_o, w_o, u_o, y_o, rmem_o = pl.pallas_call(
      _block_kernel,
      grid=grid,
      in_specs=in_specs,
      out_specs=out_specs,
      out_shape=out_shapes,
  )(rw, ww, us, free_gates, inp, qry, memory,
    I_w, I_b, W1_w, W1_b, W2_w, W2_b, R1_w, R1_b, R2_w, R2_b, O_w, O_b)
  return (wr_o, w_o, u_o, y_o, rmem_o)


# speedup vs baseline: 2.2746x; 2.2746x over previous
import functools

import jax
import jax.numpy as jnp
from jax.experimental import pallas as pl
from jax.experimental.pallas import tpu as pltpu

B, M, E, IN, Q, OUT, H = 128, 2048, 128, 512, 512, 512, 64
BB = 8
LOGM = 11
F32 = jnp.float32


def _dg(a, b, dims):
  return jax.lax.dot_general(a, b, (dims, ((), ())),
                             preferred_element_type=F32)


def _xor_partner(x, d):
  right = pltpu.roll(x, d, 1)
  left = pltpu.roll(x, M - d, 1)
  lane = jax.lax.broadcasted_iota(jnp.int32, x.shape, 1)
  return jnp.where((lane & d) == 0, left, right)


def _sort_record(keys, idx):
  lane = jax.lax.broadcasted_iota(jnp.int32, keys.shape, 1)
  records = []
  for k in range(1, LOGM + 1):
    asc = (lane & (1 << k)) == 0
    for j in range(k - 1, -1, -1):
      d = 1 << j
      ok = _xor_partner(keys, d)
      oi = _xor_partner(idx, d)
      less = (keys < ok) | ((keys == ok) & (idx < oi))
      lower = (lane & d) == 0
      take_self = less ^ lower ^ asc
      keys = jnp.where(take_self, keys, ok)
      idx = jnp.where(take_self, idx, oi)
      records.append((d, jnp.where(take_self, 1.0, 0.0).astype(F32)))
  return keys, records


def _unsort(x, records):
  for d, mask in reversed(records):
    x = jnp.where(mask > 0.5, x, _xor_partner(x, d))
  return x


def _prefix_prod(x):
  lane = jax.lax.broadcasted_iota(jnp.int32, x.shape, 1)
  for k in range(LOGM):
    s = 1 << k
    x = x * jnp.where(lane >= s, pltpu.roll(x, s, 1), 1.0)
  return x


def _lane_allreduce(x, op):
  y = x
  w = M // 2
  while w >= 128:
    y = op(y[:, :w], y[:, w:2 * w])
    w //= 2
  s = 64
  while s >= 1:
    y = op(y, pltpu.roll(y, s, 1))
    s //= 2
  return jnp.concatenate([y] * (M // 128), axis=1)


def _softmax_lanes(x):
  e = jnp.exp(x - _lane_allreduce(x, jnp.maximum))
  return e / _lane_allreduce(e, jnp.add)


def _block_kernel(rw_ref, ww_ref, us_ref, fg_ref, inp_ref, qry_ref, mem_ref,
                  iw_ref, ib_ref, w1w_ref, w1b_ref, w2w_ref, w2b_ref,
                  r1w_ref, r1b_ref, r2w_ref, r2b_ref, ow_ref, ob_ref,
                  wr_out, w_out, u_out, y_out, rmem_out):
  inp = inp_ref[...]
  qry = qry_ref[...]
  mem = mem_ref[...]
  mem2d = mem.reshape(BB * M, E)

  e_vec = _dg(inp, iw_ref[...], ((1,), (1,))) + ib_ref[...][None, :]
  wi = _dg(e_vec, w1w_ref[...], ((1,), (1,))) + w1b_ref[...][None, :]
  cvec = _dg(wi, w2w_ref[...], ((1,), (0,)))
  cconst = jnp.sum(wi * w2b_ref[...][None, :], axis=1, keepdims=True)
  rq = _dg(qry, r1w_ref[...], ((1,), (1,))) + r1b_ref[...][None, :]
  v = _dg(rq, r2w_ref[...], ((1,), (0,)))
  rconst = jnp.sum(rq * r2b_ref[...][None, :], axis=1, keepdims=True)

  cv = jnp.concatenate([cvec, v], axis=0)
  rt = _dg(cv, mem2d, ((1,), (1,)))
  mul = jnp.concatenate(
      [rt[b:b + 1, b * M:(b + 1) * M] for b in range(BB)], axis=0)
  mul2m = jnp.concatenate(
      [rt[BB + b:BB + b + 1, b * M:(b + 1) * M] for b in range(BB)], axis=0)
  mul = mul + cconst

  w_content = _softmax_lanes(mul)

  us = 1e-05 + (1.0 - 1e-05) * us_ref[...]
  ww = ww_ref[...]
  u = us + ww - us * ww
  u = u * (1.0 - fg_ref[...] * rw_ref[...])
  u_out[...] = u

  idx = jax.lax.broadcasted_iota(jnp.int32, u.shape, 1)
  su, recs = _sort_record(u, idx)
  alloc_sorted = (1.0 - su) * _prefix_prod(su)
  alloc = _unsort(alloc_sorted, recs)
  w = (w_content + alloc) * 0.5
  w_out[...] = w

  ev = jnp.sum(e_vec * v, axis=1, keepdims=True)
  mul2 = mul2m + w * ev + rconst
  wr = _softmax_lanes(mul2)
  wr_out[...] = wr

  rmem = mem + w[:, :, None] * e_vec[:, None, :]
  rmem_out[...] = rmem

  lane16k = jax.lax.broadcasted_iota(jnp.int32, (BB, BB * M), 1)
  wr_bd = jnp.where((lane16k >> LOGM) == jax.lax.broadcasted_iota(
      jnp.int32, (BB, BB * M), 0),
      jnp.concatenate([wr] * BB, axis=1), 0.0)
  outv = _dg(wr_bd, rmem.reshape(BB * M, E), ((1,), (0,)))
  y_out[...] = _dg(outv, ow_ref[...], ((1,), (1,))) + ob_ref[...][None, :]


def _full(shape):
  nd = len(shape)
  return pl.BlockSpec(shape, lambda i: (0,) * nd)


def _blk(shape):
  nd = len(shape)
  return pl.BlockSpec(shape, lambda i: (i,) + (0,) * (nd - 1))


@jax.jit
def kernel(state_vector, free_gates, input, query, memory, I_w, I_b, W1_w,
           W1_b, W2_w, W2_b, R1_w, R1_b, R2_w, R2_b, O_w, O_b):
  rw, ww, us = state_vector[0], state_vector[1], state_vector[2]
  inp = input[:, 0, :]
  qry = query[:, 0, :]

  grid = (B // BB,)
  out_shapes = (
      jax.ShapeDtypeStruct((B, M), jnp.float32),
      jax.ShapeDtypeStruct((B, M), jnp.float32),
      jax.ShapeDtypeStruct((B, M), jnp.float32),
      jax.ShapeDtypeStruct((B, OUT), jnp.float32),
      jax.ShapeDtypeStruct((B, M, E), jnp.float32),
  )
  in_specs = [
      _blk((BB, M)), _blk((BB, M)), _blk((BB, M)),
      _blk((BB, 1)),
      _blk((BB, IN)), _blk((BB, Q)),
      _blk((BB, M, E)),
      _full((E, IN)), _full((E,)),
      _full((H, E)), _full((H,)),
      _full((H, E)), _full((H,)),
      _full((H, Q)), _full((H,)),
      _full((H, E)), _full((H,)),
      _full((OUT, E)), _full((OUT,)),
  ]
  out_specs = (
      _blk((BB, M)), _blk((BB, M)), _blk((BB, M)),
      _blk((BB, OUT)),
      _blk((BB, M, E)),
  )
  wr_o, w_o, u_o, y_o, rmem_o = pl.pallas_call(
      _block_kernel,
      grid=grid,
      in_specs=in_specs,
      out_specs=out_specs,
      out_shape=out_shapes,
  )(rw, ww, us, free_gates, inp, qry, memory,
    I_w, I_b, W1_w, W1_b, W2_w, W2_b, R1_w, R1_b, R2_w, R2_b, O_w, O_b)
  return (wr_o, w_o, u_o, y_o, rmem_o)
